# baseline (speedup 1.0000x reference)
import functools

import jax
import jax.numpy as jnp
import numpy as np
from jax.experimental import pallas as pl
from jax.experimental.pallas import tpu as pltpu
from jax.experimental.pallas import tpu_sc as plsc

_B, _T, _D, _C = 10, 2048, 2048, 20
_K = 292
_DT = 512
_ND = _D // _DT
_NPAIR = 3
_INT_MIN = -(2 ** 31)
_M31 = 2 ** 31 - 1
_NW = 32
_NU = _B * 2 * (_C + 1)
_LANES = 16


def _prep_kernel(cas_ref, atn_ref, mask_ref, v_ref, f_ref,
                 scal_ref, a_ref, at_ref, mn_ref):
    mutual = jnp.float32(0.0)
    norm_a = jnp.float32(0.0)
    norm_v = jnp.float32(0.0)
    norm_f = jnp.float32(0.0)
    guide_a = jnp.float32(0.0)
    guide_v = jnp.float32(0.0)
    guide_f = jnp.float32(0.0)
    inv_t = jnp.float32(1.0 / _T)
    for i in range(_B):
        cas = cas_ref[i]
        atn = atn_ref[i]
        msk = mask_ref[i]
        v = v_ref[i] * msk
        f = f_ref[i] * msk
        el = cas * msk
        atn_m = atn * msk
        mutual += jnp.mean((v - f) ** 2)
        mn = jnp.min(el, axis=0, keepdims=True)
        at_ref[i] = jnp.concatenate([jnp.ones((1, _T), jnp.float32), atn_m],
                                    axis=0)
        mn_ref[i] = jnp.concatenate([jnp.zeros((1, _T), jnp.float32), mn],
                                    axis=0)
        mx = jnp.max(el, axis=0, keepdims=True)
        z = jnp.sum(jnp.exp(el - mx), axis=0, keepdims=True)
        bg = jnp.exp(el[_C:_C + 1, :] - mx) / z
        norm_a += jnp.sum(atn_m) * inv_t
        norm_v += jnp.sum(v) * inv_t
        norm_f += jnp.sum(f) * inv_t
        guide_a += jnp.sum(jnp.abs(1.0 - atn_m - bg)) * inv_t
        guide_v += jnp.sum(jnp.abs(1.0 - v - bg)) * inv_t
        guide_f += jnp.sum(jnp.abs(1.0 - f - bg)) * inv_t
        if i < 2 * _NPAIR:
            mnu = jnp.min(cas, axis=0, keepdims=True)
            st = atn * (cas - mnu) + mnu
            mxt = jnp.max(st, axis=1, keepdims=True)
            e = jnp.exp(st - mxt)
            zt = jnp.sum(e, axis=1, keepdims=True)
            a1 = e / zt
            al = (1.0 - a1) * jnp.float32(1.0 / (_T - 1))
            pad = jnp.zeros((3, _T), jnp.float32)
            a_ref[i] = jnp.concatenate([a1, pad, al, pad], axis=0)
    packed = jnp.concatenate(
        [jnp.reshape(s, (1, 1)) for s in
         (mutual, norm_a, norm_v, norm_f, guide_a, guide_v, guide_f)]
        + [jnp.zeros((1, 9), jnp.float32)],
        axis=1)
    scal_ref[...] = packed


def _sc_topk_body(cas_hbm, mask_hbm, at_hbm, mn_hbm, out_hbm,
                  colv, maskv, atv, mnv, valsv, keysv, resv, sem):
    c_ax = jax.lax.axis_index("c")
    s_ax = jax.lax.axis_index("s")
    wid = s_ax * 2 + c_ax
    kf = jnp.float32(_K)
    ki = jnp.int32(_K)
    lanes = jax.lax.iota(jnp.int32, _LANES)
    resv[...] = jnp.zeros((_LANES,), jnp.float32)
    for k in range(14):
        u = wid + _NW * k

        @pl.when(u < _NU)
        def _unit():
            i = u // 42
            r = u % 42
            var = r // 21
            cls = r % 21
            c1 = pltpu.async_copy(cas_hbm.at[i, cls], colv, sem)
            c2 = pltpu.async_copy(mask_hbm.at[i], maskv, sem)
            c3 = pltpu.async_copy(at_hbm.at[i, var], atv, sem)
            c4 = pltpu.async_copy(mn_hbm.at[i, var], mnv, sem)
            c1.wait()
            c2.wait()
            c3.wait()
            c4.wait()

            @plsc.parallel_loop(0, _T // _LANES, 1, unroll=8)
            def _build(j):
                sl = pl.ds(j * _LANES, _LANES)
                x = atv[sl] * (colv[sl] * maskv[sl] - mnv[sl]) + mnv[sl]
                valsv[sl] = x
                b = jax.lax.bitcast_convert_type(x, jnp.int32)
                keysv[sl] = jnp.where(b < 0, b ^ _M31, b)

            def cnt_ge(cand):
                def body(j, acc):
                    kv = keysv[pl.ds(j * _LANES, _LANES)]
                    return acc + plsc.all_reduce_population_count(kv >= cand)
                return plsc.parallel_loop(
                    0, _T // _LANES, 1, unroll=8,
                    carry=jnp.zeros((_LANES,), jnp.int32))(body)

            zero_v = jnp.zeros((_LANES,), jnp.int32)
            t0 = jnp.where(cnt_ge(zero_v) >= ki, zero_v,
                           jnp.full((_LANES,), _INT_MIN, jnp.int32))

            def search(b, t):
                bit = jax.lax.shift_left(jnp.int32(1), jnp.int32(30) - b)
                cand = t + bit
                return jnp.where(cnt_ge(cand) >= ki, cand, t)

            t = jax.lax.fori_loop(jnp.int32(0), jnp.int32(31), search, t0)

            def sumpass(j, carry):
                cg, ss = carry
                sl = pl.ds(j * _LANES, _LANES)
                m = keysv[sl] > t
                cg = cg + plsc.all_reduce_population_count(m)
                ss = ss + jnp.where(m, valsv[sl], jnp.float32(0.0))
                return (cg, ss)

            cg, ss = plsc.parallel_loop(
                0, _T // _LANES, 1, unroll=8,
                carry=(jnp.zeros((_LANES,), jnp.int32),
                       jnp.zeros((_LANES,), jnp.float32)))(sumpass)
            ssum = jnp.sum(ss)
            tval = jax.lax.bitcast_convert_type(
                jnp.where(t < 0, t ^ _M31, t), jnp.float32)
            res = (ssum + (kf - cg.astype(jnp.float32)) * tval) \
                * jnp.float32(1.0 / _K)
            resv[...] = resv[...] + jnp.where(lanes == k, res,
                                              jnp.float32(0.0))
    pltpu.sync_copy(resv, out_hbm.at[pl.ds(wid * _LANES, _LANES)])


def _sc_topk(cas_t, mask2, at_stack, mn_stack):
    mesh = plsc.VectorSubcoreMesh(core_axis_name="c", subcore_axis_name="s")
    fn = pl.kernel(
        _sc_topk_body,
        mesh=mesh,
        compiler_params=pltpu.CompilerParams(needs_layout_passes=False),
        out_type=jax.ShapeDtypeStruct((_NW * _LANES,), jnp.float32),
        scratch_types=[
            pltpu.VMEM((_T,), jnp.float32),
            pltpu.VMEM((_T,), jnp.float32),
            pltpu.VMEM((_T,), jnp.float32),
            pltpu.VMEM((_T,), jnp.float32),
            pltpu.VMEM((_T,), jnp.float32),
            pltpu.VMEM((_T,), jnp.int32),
            pltpu.VMEM((_LANES,), jnp.float32),
            pltpu.SemaphoreType.DMA,
        ],
    )
    return fn(cas_t, mask2, at_stack, mn_stack)


def _contrast_kernel(a1_ref, a2_ref, x1_ref, x2_ref, l1_ref, l2_ref,
                     out_ref, acc_ref):
    d = pl.program_id(1)

    @pl.when(d == 0)
    def _zero():
        acc_ref[...] = jnp.zeros_like(acc_ref)

    dn = (((1,), (0,)), ((), ()))
    m1 = jax.lax.dot_general(a1_ref[0], x1_ref[0], dn,
                             preferred_element_type=jnp.float32,
                             precision=jax.lax.Precision.HIGHEST)
    m2 = jax.lax.dot_general(a2_ref[0], x2_ref[0], dn,
                             preferred_element_type=jnp.float32,
                             precision=jax.lax.Precision.HIGHEST)
    h1 = m1[0:_C + 1]
    l1 = m1[24:24 + _C + 1]
    h2 = m2[0:_C + 1]
    l2 = m2[24:24 + _C + 1]
    acc_ref[0] = acc_ref[0] + h1 * h2
    acc_ref[1] = acc_ref[1] + h1 * l2
    acc_ref[2] = acc_ref[2] + h2 * l1
    acc_ref[3] = acc_ref[3] + h1 * h1
    acc_ref[4] = acc_ref[4] + h2 * h2
    acc_ref[5] = acc_ref[5] + l1 * l1
    acc_ref[6] = acc_ref[6] + l2 * l2

    @pl.when(d == _ND - 1)
    def _final():
        h1h2 = jnp.sum(acc_ref[0], axis=1, keepdims=True)
        h1l2 = jnp.sum(acc_ref[1], axis=1, keepdims=True)
        h2l1 = jnp.sum(acc_ref[2], axis=1, keepdims=True)
        nh1 = jnp.sqrt(jnp.sum(acc_ref[3], axis=1, keepdims=True))
        nh2 = jnp.sqrt(jnp.sum(acc_ref[4], axis=1, keepdims=True))
        nl1 = jnp.sqrt(jnp.sum(acc_ref[5], axis=1, keepdims=True))
        nl2 = jnp.sqrt(jnp.sum(acc_ref[6], axis=1, keepdims=True))
        d1 = 1.0 - h1h2 / (nh1 * nh2)
        d2 = 1.0 - h1l2 / (nh1 * nl2)
        d3 = 1.0 - h2l1 / (nh2 * nl1)
        ll = l1_ref[0] * l2_ref[0]
        part = 0.5 * (jnp.sum(jnp.maximum(d1 - d2 + 0.5, 0.0) * ll)
                      + jnp.sum(jnp.maximum(d1 - d3 + 0.5, 0.0) * ll))
        ntmp = jnp.sum(ll)
        out_ref[0] = jnp.concatenate(
            [jnp.reshape(part, (1, 1)), jnp.reshape(ntmp, (1, 1))], axis=1)


def _final_kernel(il_ref, lwb_ref, scal_ref, pairs_ref, out_ref):
    il = il_ref[...]
    lwb = lwb_ref[...]
    lwbn = lwb / (jnp.sum(lwb, axis=1, keepdims=True) + 1e-4)
    mx = jnp.max(il, axis=1, keepdims=True)
    lz = jnp.log(jnp.sum(jnp.exp(il - mx), axis=1, keepdims=True))
    ls = il - mx - lz
    milr = -jnp.sum(lwbn * ls, axis=1, keepdims=True)
    mil_orig = jnp.sum(milr[0:_B]) * jnp.float32(0.1)
    mil_supp = jnp.sum(milr[_B:2 * _B]) * jnp.float32(0.1)
    p = pairs_ref[...]
    contr = jnp.sum(p[:, :, 0]) / jnp.sum(p[:, :, 1])
    s = scal_ref[...]
    mutual = s[0, 0] * jnp.float32(0.1)
    norm_avg = (s[0, 1] + s[0, 2] + s[0, 3]) * jnp.float32(0.1 / 3.0)
    guide_avg = (s[0, 4] + s[0, 5] + s[0, 6]) * jnp.float32(0.1 / 3.0)
    total = (mil_orig + mil_supp + contr + mutual
             + 0.8 * norm_avg + 0.8 * guide_avg)
    out_ref[...] = jnp.concatenate(
        [jnp.reshape(x, (1, 1)) for x in
         (total, mil_orig, mil_supp, contr, mutual, norm_avg, guide_avg)]
        + [jnp.zeros((1, 1), jnp.float32)], axis=1)


_UNSCRAMBLE = np.arange(_NU, dtype=np.int32) % _NW * _LANES \
    + np.arange(_NU, dtype=np.int32) // _NW


def kernel(feat, cas, attn, mask, v_atn, f_atn, labels):
    f32 = jnp.float32
    cas_t = jnp.transpose(cas, (0, 2, 1))
    atn_t = jnp.transpose(attn, (0, 2, 1))
    mask_t = jnp.transpose(mask, (0, 2, 1))
    v_t = jnp.transpose(v_atn, (0, 2, 1))
    f_t = jnp.transpose(f_atn, (0, 2, 1))
    labb = jnp.concatenate([labels, jnp.ones_like(labels[:, :1])], axis=1)
    labs = jnp.concatenate([labels, jnp.zeros_like(labels[:, :1])], axis=1)
    labs3 = labs[:, :, None]

    scal, amats, at_stack, mn_stack = pl.pallas_call(
        _prep_kernel,
        out_shape=(jax.ShapeDtypeStruct((1, 16), f32),
                   jax.ShapeDtypeStruct((2 * _NPAIR, 48, _T), f32),
                   jax.ShapeDtypeStruct((_B, 2, _T), f32),
                   jax.ShapeDtypeStruct((_B, 2, _T), f32)),
    )(cas_t, atn_t, mask_t, v_t, f_t)

    ilvec = _sc_topk(cas_t, mask[:, :, 0], at_stack, mn_stack)

    pairs = pl.pallas_call(
        _contrast_kernel,
        grid=(_NPAIR, _ND),
        in_specs=[
            pl.BlockSpec((1, 48, _T), lambda p, d: (2 * p, 0, 0)),
            pl.BlockSpec((1, 48, _T), lambda p, d: (2 * p + 1, 0, 0)),
            pl.BlockSpec((1, _T, _DT), lambda p, d: (2 * p, 0, d)),
            pl.BlockSpec((1, _T, _DT), lambda p, d: (2 * p + 1, 0, d)),
            pl.BlockSpec((1, _C + 1, 1), lambda p, d: (2 * p, 0, 0)),
            pl.BlockSpec((1, _C + 1, 1), lambda p, d: (2 * p + 1, 0, 0)),
        ],
        out_specs=pl.BlockSpec((1, 1, 2), lambda p, d: (p, 0, 0)),
        out_shape=jax.ShapeDtypeStruct((_NPAIR, 1, 2), f32),
        scratch_shapes=[pltpu.VMEM((7, _C + 1, _DT), f32)],
    )(amats, amats, feat, feat, labs3, labs3)

    il20 = jnp.take(ilvec, _UNSCRAMBLE).reshape(_B, 2, _C + 1)
    il20 = jnp.concatenate([il20[:, 0, :], il20[:, 1, :]], axis=0)
    lwb20 = jnp.concatenate([labb, labs], axis=0)

    out8 = pl.pallas_call(
        _final_kernel,
        out_shape=jax.ShapeDtypeStruct((1, 8), f32),
    )(il20, lwb20, scal, pairs)

    o = out8[0]
    return (o[0], o[1], o[2], o[3], o[4], o[5], o[6])


# speedup vs baseline: 1.4695x; 1.4695x over previous
import jax
import jax.numpy as jnp
from jax.experimental import pallas as pl
from jax.experimental.pallas import tpu as pltpu

_B, _T, _D, _C = 10, 2048, 2048, 20
_K = 292
_DT = 512
_ND = _D // _DT
_NPAIR = 3
_RP = 24
_INT_MIN = -(2 ** 31)
_M31 = 2 ** 31 - 1

_PASS_SCHED = {s: list(range(30 - 3 * (s - 1), 30 - 3 * s, -1))
               for s in range(1, 10)}
_PASS_SCHED[10] = [3, 2, 1, 0]


def _sortable(bits):
    return jnp.where(bits < 0, bits ^ _M31, bits)


def _mil(il, lab):
    lwb = lab * (1.0 / (jnp.sum(lab) + 1e-4))
    mx = jnp.max(il)
    ls = il - mx - jnp.log(jnp.sum(jnp.exp(il - mx)))
    return -jnp.sum(lwb * ls)


def _fused_kernel(cas_ref, atn_ref, mask_ref, v_ref, f_ref, labb_ref,
                  labs_ref, x1_ref, x2_ref, scal_ref, milv_ref, pairs_ref,
                  stage_ref, valt_ref, keyt_ref, t_ref, am_ref, acc_ref):
    p = pl.program_id(0)
    d = pl.program_id(1)
    sidx = p * _ND + d
    kf = jnp.float32(_K)

    def cnt_ge(c):
        parts = []
        for j in range(16):
            blk = keyt_ref[pl.ds(128 * j, 128), :]
            parts.append(jnp.sum((blk >= c).astype(jnp.float32),
                                 axis=0, keepdims=True))
        while len(parts) > 1:
            parts = [a + b for a, b in zip(parts[0::2], parts[1::2])]
        return parts[0]

    @pl.when(sidx == 0)
    def _build():
        mutual = jnp.float32(0.0)
        norm_a = jnp.float32(0.0)
        norm_v = jnp.float32(0.0)
        norm_f = jnp.float32(0.0)
        guide_a = jnp.float32(0.0)
        guide_v = jnp.float32(0.0)
        guide_f = jnp.float32(0.0)
        inv_t = jnp.float32(1.0 / _T)
        pad = jnp.full((_RP - _C - 1, _T), -jnp.inf, jnp.float32)
        for i in range(_B):
            cas = cas_ref[i]
            atn = atn_ref[i]
            msk = mask_ref[i]
            v = v_ref[i] * msk
            f = f_ref[i] * msk
            el = cas * msk
            atn_m = atn * msk
            mutual += jnp.mean((v - f) ** 2)
            mn = jnp.min(el, axis=0, keepdims=True)
            supp = atn_m * (el - mn) + mn
            stage_ref[pl.ds(_RP * i, _RP), :] = \
                jnp.concatenate([el, pad], axis=0)
            stage_ref[pl.ds(_RP * (_B + i), _RP), :] = \
                jnp.concatenate([supp, pad], axis=0)
            mx = jnp.max(el, axis=0, keepdims=True)
            z = jnp.sum(jnp.exp(el - mx), axis=0, keepdims=True)
            bg = jnp.exp(el[_C:_C + 1, :] - mx) / z
            norm_a += jnp.sum(atn_m) * inv_t
            norm_v += jnp.sum(v) * inv_t
            norm_f += jnp.sum(f) * inv_t
            guide_a += jnp.sum(jnp.abs(1.0 - atn_m - bg)) * inv_t
            guide_v += jnp.sum(jnp.abs(1.0 - v - bg)) * inv_t
            guide_f += jnp.sum(jnp.abs(1.0 - f - bg)) * inv_t
        ipad = jnp.full((8, _T), -jnp.inf, jnp.float32)
        for g in range(4):
            blk = jnp.concatenate(
                [stage_ref[pl.ds(120 * g, 120), :], ipad], axis=0)
            valt_ref[:, 128 * g:128 * (g + 1)] = jnp.transpose(blk, (1, 0))
        keyt_ref[...] = _sortable(
            jax.lax.bitcast_convert_type(valt_ref[...], jnp.int32))
        zero = jnp.zeros((1, 512), jnp.int32)
        t_ref[...] = jnp.where(cnt_ge(zero) >= kf, zero,
                               jnp.full((1, 512), _INT_MIN, jnp.int32))
        scal_ref[...] = jnp.concatenate(
            [jnp.reshape(s, (1, 1)) for s in
             (mutual, norm_a, norm_v, norm_f, guide_a, guide_v, guide_f)]
            + [jnp.zeros((1, 9), jnp.float32)], axis=1)

    for s, bits in _PASS_SCHED.items():
        @pl.when(sidx == s)
        def _passes(bits=bits):
            for b in bits:
                t = t_ref[...]
                cand = t + jnp.int32(1 << b)
                t_ref[...] = jnp.where(cnt_ge(cand) >= kf, cand, t)

    @pl.when(d == 0)
    def _amats():
        acc_ref[...] = jnp.zeros_like(acc_ref)
        for q in range(2):
            idx = 2 * p + q
            cas = cas_ref[idx]
            atn = atn_ref[idx]
            mnu = jnp.min(cas, axis=0, keepdims=True)
            st = atn * (cas - mnu) + mnu
            mxt = jnp.max(st, axis=1, keepdims=True)
            e = jnp.exp(st - mxt)
            zt = jnp.sum(e, axis=1, keepdims=True)
            a1 = e / zt
            al = (1.0 - a1) * jnp.float32(1.0 / (_T - 1))
            zpad = jnp.zeros((3, _T), jnp.float32)
            am_ref[q] = jnp.concatenate([a1, zpad, al, zpad], axis=0)

    dn = (((1,), (0,)), ((), ()))
    m1 = jax.lax.dot_general(am_ref[0], x1_ref[0], dn,
                             preferred_element_type=jnp.float32,
                             precision=jax.lax.Precision.DEFAULT)
    m2 = jax.lax.dot_general(am_ref[1], x2_ref[0], dn,
                             preferred_element_type=jnp.float32,
                             precision=jax.lax.Precision.DEFAULT)
    h1 = m1[0:_C + 1]
    l1 = m1[24:24 + _C + 1]
    h2 = m2[0:_C + 1]
    l2 = m2[24:24 + _C + 1]
    acc_ref[0] = acc_ref[0] + h1 * h2
    acc_ref[1] = acc_ref[1] + h1 * l2
    acc_ref[2] = acc_ref[2] + h2 * l1
    acc_ref[3] = acc_ref[3] + h1 * h1
    acc_ref[4] = acc_ref[4] + h2 * h2
    acc_ref[5] = acc_ref[5] + l1 * l1
    acc_ref[6] = acc_ref[6] + l2 * l2

    @pl.when(d == _ND - 1)
    def _pair_final():
        h1h2 = jnp.sum(acc_ref[0], axis=1, keepdims=True)
        h1l2 = jnp.sum(acc_ref[1], axis=1, keepdims=True)
        h2l1 = jnp.sum(acc_ref[2], axis=1, keepdims=True)
        nh1 = jnp.sqrt(jnp.sum(acc_ref[3], axis=1, keepdims=True))
        nh2 = jnp.sqrt(jnp.sum(acc_ref[4], axis=1, keepdims=True))
        nl1 = jnp.sqrt(jnp.sum(acc_ref[5], axis=1, keepdims=True))
        nl2 = jnp.sqrt(jnp.sum(acc_ref[6], axis=1, keepdims=True))
        d1 = 1.0 - h1h2 / (nh1 * nh2)
        d2 = 1.0 - h1l2 / (nh1 * nl2)
        d3 = 1.0 - h2l1 / (nh2 * nl1)
        ll = labs_ref[2 * p] * labs_ref[2 * p + 1]
        part = 0.5 * (jnp.sum(jnp.maximum(d1 - d2 + 0.5, 0.0) * ll)
                      + jnp.sum(jnp.maximum(d1 - d3 + 0.5, 0.0) * ll))
        ntmp = jnp.sum(ll)
        pairs_ref[0] = jnp.concatenate(
            [jnp.reshape(part, (1, 1)), jnp.reshape(ntmp, (1, 1))], axis=1)

    @pl.when(sidx == _NPAIR * _ND - 1)
    def _finalize():
        t = t_ref[...]
        cparts = []
        sparts = []
        for j in range(16):
            kblk = keyt_ref[pl.ds(128 * j, 128), :]
            vblk = valt_ref[pl.ds(128 * j, 128), :]
            gtb = kblk > t
            cparts.append(jnp.sum(gtb.astype(jnp.float32),
                                  axis=0, keepdims=True))
            sparts.append(jnp.sum(jnp.where(gtb, vblk, jnp.float32(0.0)),
                                  axis=0, keepdims=True))
        while len(cparts) > 1:
            cparts = [a + b for a, b in zip(cparts[0::2], cparts[1::2])]
            sparts = [a + b for a, b in zip(sparts[0::2], sparts[1::2])]
        tval = jax.lax.bitcast_convert_type(_sortable(t), jnp.float32)
        il_vec = (sparts[0] + (kf - cparts[0]) * tval) \
            * jnp.float32(1.0 / _K)
        il_col = jnp.transpose(il_vec, (1, 0))
        mil_orig = jnp.float32(0.0)
        mil_supp = jnp.float32(0.0)
        for i in range(_B):
            ge, je = i // 5, i % 5
            il_el = il_col[128 * ge + _RP * je:128 * ge + _RP * je + _C + 1]
            gs, js = (_B + i) // 5, (_B + i) % 5
            il_sp = il_col[128 * gs + _RP * js:128 * gs + _RP * js + _C + 1]
            mil_orig += _mil(il_el, labb_ref[i])
            mil_supp += _mil(il_sp, labs_ref[i])
        milv_ref[...] = jnp.concatenate(
            [jnp.reshape(mil_orig, (1, 1)), jnp.reshape(mil_supp, (1, 1)),
             jnp.zeros((1, 6), jnp.float32)], axis=1)


def kernel(feat, cas, attn, mask, v_atn, f_atn, labels):
    f32 = jnp.float32
    cas_t = jnp.transpose(cas, (0, 2, 1))
    atn_t = jnp.transpose(attn, (0, 2, 1))
    mask_t = jnp.transpose(mask, (0, 2, 1))
    v_t = jnp.transpose(v_atn, (0, 2, 1))
    f_t = jnp.transpose(f_atn, (0, 2, 1))
    labb = jnp.concatenate([labels, jnp.ones_like(labels[:, :1])], axis=1)[:, :, None]
    labs = jnp.concatenate([labels, jnp.zeros_like(labels[:, :1])], axis=1)[:, :, None]

    full = lambda shape: pl.BlockSpec(shape, lambda p, d: (0,) * len(shape))
    scal, milv, pairs = pl.pallas_call(
        _fused_kernel,
        grid=(_NPAIR, _ND),
        in_specs=[
            full((_B, _C + 1, _T)),
            full((_B, 1, _T)),
            full((_B, 1, _T)),
            full((_B, 1, _T)),
            full((_B, 1, _T)),
            full((_B, _C + 1, 1)),
            full((_B, _C + 1, 1)),
            pl.BlockSpec((1, _T, _DT), lambda p, d: (2 * p, 0, d)),
            pl.BlockSpec((1, _T, _DT), lambda p, d: (2 * p + 1, 0, d)),
        ],
        out_specs=(pl.BlockSpec((1, 16), lambda p, d: (0, 0)),
                   pl.BlockSpec((1, 8), lambda p, d: (0, 0)),
                   pl.BlockSpec((1, 1, 2), lambda p, d: (p, 0, 0))),
        out_shape=(jax.ShapeDtypeStruct((1, 16), f32),
                   jax.ShapeDtypeStruct((1, 8), f32),
                   jax.ShapeDtypeStruct((_NPAIR, 1, 2), f32)),
        scratch_shapes=[pltpu.VMEM((2 * _B * _RP, _T), f32),
                        pltpu.VMEM((_T, 512), f32),
                        pltpu.VMEM((_T, 512), jnp.int32),
                        pltpu.VMEM((1, 512), jnp.int32),
                        pltpu.VMEM((2, 48, _T), f32),
                        pltpu.VMEM((7, _C + 1, _DT), f32)],
    )(cas_t, atn_t, mask_t, v_t, f_t, labb, labs, feat, feat)

    loss_contrastive = jnp.sum(pairs[:, 0, 0]) / jnp.sum(pairs[:, 0, 1])
    s = scal[0]
    inv = f32(0.1)
    mil_orig = milv[0, 0] * inv
    mil_supp = milv[0, 1] * inv
    mutual = s[0] * inv
    norm_avg = (s[1] + s[2] + s[3]) * (inv / 3.0)
    guide_avg = (s[4] + s[5] + s[6]) * (inv / 3.0)
    total = (mil_orig + mil_supp + loss_contrastive + mutual
             + 0.8 * norm_avg + 0.8 * guide_avg)
    return (total, mil_orig, mil_supp, loss_contrastive, mutual,
            norm_avg, guide_avg)
